# baseline (speedup 1.0000x reference)
import functools

import jax
import jax.numpy as jnp
from jax import lax
from jax.experimental import pallas as pl
from jax.experimental.pallas import tpu as pltpu
from jax.experimental.pallas import tpu_sc as plsc

_NC = 2
_NS = 16
_K = 125


def _deg_rs(dst16, npad, h):
  erows = dst16.shape[0]
  rpt = npad // _NS
  ept = erows // _NS
  mesh = plsc.VectorSubcoreMesh(core_axis_name="c", subcore_axis_name="s")

  @functools.partial(
      pl.kernel,
      out_type=jax.ShapeDtypeStruct((npad, h), jnp.float32),
      mesh=mesh,
      compiler_params=pltpu.CompilerParams(use_tc_tiling_on_sc=False,
                                           needs_layout_passes=False),
      scratch_types=[
          pltpu.VMEM_SHARED((_NS, npad), jnp.float32),
          pltpu.VMEM((npad,), jnp.float32),
          pltpu.VMEM((ept, 16), jnp.int32),
          pltpu.VMEM((_NS, rpt), jnp.float32),
          pltpu.VMEM((rpt, h), jnp.float32),
          pltpu.SemaphoreType.DMA,
      ],
  )
  def kern(dst_hbm, out_hbm, shared, hist, dstv, sumv, rsb, sem):
    c = lax.axis_index("c")
    s = lax.axis_index("s")
    r0 = s * rpt
    cp = pltpu.async_copy(dst_hbm.at[pl.ds(s * ept, ept)], dstv, sem)

    def zfn(i, carry):
      hist[pl.ds(16 * i, 16)] = jnp.zeros((16,), jnp.float32)
      return carry

    lax.fori_loop(0, npad // 16, zfn, 0)
    cp.wait()

    ones = jnp.ones((16,), jnp.float32)

    def afn(r, carry):
      plsc.addupdate_scatter(hist, [dstv[r]], ones)
      return carry

    lax.fori_loop(0, ept, afn, 0)
    pltpu.sync_copy(hist, shared.at[s])
    plsc.subcore_barrier()

    for t in range(_NS):
      pltpu.async_copy(shared.at[t, pl.ds(r0, rpt)], sumv.at[t], sem)
    for t in range(_NS):
      pltpu.make_async_copy(shared.at[t, pl.ds(r0, rpt)], sumv.at[t],
                            sem).wait()

    def chunk_fn(q, carry):
      col = 16 * q
      tot = sumv[0, pl.ds(col, 16)]
      for t in range(1, _NS):
        tot = tot + sumv[t, pl.ds(col, 16)]
      deg = tot + 1.0
      i = plsc.bitcast(deg, jnp.int32)
      i = jnp.int32(0x5F3759DF) - lax.shift_right_logical(i, 1)
      y = plsc.bitcast(i, jnp.float32)
      y = y * (1.5 - 0.5 * deg * y * y)
      y = y * (1.5 - 0.5 * deg * y * y)
      y = y * (1.5 - 0.5 * deg * y * y)
      for l in range(16):
        val = jnp.full((16,), y[l])
        for q2 in range(h // 16):
          rsb[col + l, pl.ds(16 * q2, 16)] = val
      return carry

    lax.fori_loop(0, rpt // 16, chunk_fn, 0)

    @pl.when(c == 0)
    def _():
      pltpu.sync_copy(rsb, out_hbm.at[pl.ds(r0, rpt)])

  return kern(dst16)


def _seg_sum(src2d, dst2d, g, zeros):
  ec, k = src2d.shape
  ng, d = g.shape
  n = zeros.shape[0]
  rpt = n // _NS
  cpt = ec // (_NC * _NS)
  mesh = plsc.VectorSubcoreMesh(core_axis_name="c", subcore_axis_name="s")

  @functools.partial(
      pl.kernel,
      out_type=jax.ShapeDtypeStruct((_NC, n, d), jnp.float32),
      mesh=mesh,
      compiler_params=pltpu.CompilerParams(use_tc_tiling_on_sc=False),
      scratch_types=[
          pltpu.VMEM_SHARED((n, d), jnp.float32),
          pltpu.VMEM((cpt, k), jnp.int32),
          pltpu.VMEM((cpt, k), jnp.int32),
          [pltpu.VMEM((k, d), jnp.float32)] * 4,
          [pltpu.SemaphoreType.DMA] * 4,
          [pltpu.SemaphoreType.DMA] * 4,
      ],
  )
  def kern(src_hbm, dst_hbm, g_hbm, z_hbm, out_hbm, acc, srcv, dstv,
           bufs, gsems, ssems):
    c = lax.axis_index("c")
    s = lax.axis_index("s")
    r0 = s * rpt
    pltpu.sync_copy(z_hbm.at[pl.ds(r0, rpt)], acc.at[pl.ds(r0, rpt)])
    row0 = c * (ec // _NC) + s * cpt
    pltpu.sync_copy(src_hbm.at[pl.ds(row0, cpt)], srcv)
    pltpu.sync_copy(dst_hbm.at[pl.ds(row0, cpt)], dstv)
    plsc.subcore_barrier()

    def gather(j, b):
      pltpu.async_copy(g_hbm.at[srcv.at[j]], bufs[b], gsems[b])

    def wait_gather(j, b):
      pltpu.make_async_copy(g_hbm.at[srcv.at[j]], bufs[b], gsems[b]).wait()

    def scatter(j, b):
      pltpu.async_copy(bufs[b], acc.at[dstv.at[j]], ssems[b], add=True)

    def wait_scatter(b):
      pltpu.make_async_copy(bufs[b], acc.at[dstv.at[0]], ssems[b]).wait()

    for b in range(4):
      gather(b, b)

    def body(i, carry):
      for b in range(4):
        j = 4 * i + b
        wait_gather(j, b)
        scatter(j, b)
        bp = (b + 3) % 4
        jprev = j - 1

        @pl.when(jnp.logical_and(jprev >= 0, jprev + 4 < cpt))
        def _():
          wait_scatter(bp)
          gather(jprev + 4, bp)

      return carry

    lax.fori_loop(0, cpt // 4, body, 0)
    for b in range(4):
      wait_scatter(b)
    plsc.subcore_barrier()
    pltpu.sync_copy(acc.at[pl.ds(r0, rpt)], out_hbm.at[c, pl.ds(r0, rpt)])

  return kern(src2d, dst2d, g, zeros)


def _layer_in(xp, rsp, w1blk):
  m, df2 = xp.shape
  w = w1blk.shape[1]
  br = m
  grid = (m // br,)

  def body(x_ref, rs_ref, w_ref, g_ref):
    hh = jnp.dot(x_ref[...], w_ref[...], preferred_element_type=jnp.float32)
    g_ref[...] = hh * rs_ref[...]

  blk = pl.BlockSpec((br, w), lambda i: (i, 0))
  return pl.pallas_call(
      body,
      grid=grid,
      in_specs=[
          pl.BlockSpec((br, df2), lambda i: (i, 0)),
          blk,
          pl.BlockSpec((df2, w), lambda i: (0, 0)),
      ],
      out_specs=blk,
      out_shape=jax.ShapeDtypeStruct((m, w), jnp.float32),
  )(xp, rsp, w1blk)


def _layer_mid(sp_pk, g1, rs, b1p, w2blk):
  m, w = g1.shape
  br = m
  grid = (m // br,)

  def body(sp_ref, g_ref, rs_ref, b_ref, w_ref, out_ref):
    rsv = rs_ref[...]
    a = jnp.maximum(rsv * (sp_ref[0] + sp_ref[1] + g_ref[...])
                    + b_ref[...][None, :], 0.0)
    out_ref[...] = jnp.dot(a, w_ref[...],
                           preferred_element_type=jnp.float32) * rsv

  blk = pl.BlockSpec((br, w), lambda i: (i, 0))
  return pl.pallas_call(
      body,
      grid=grid,
      in_specs=[pl.BlockSpec((_NC, br, w), lambda i: (0, i, 0)),
                blk, blk,
                pl.BlockSpec((w,), lambda i: (0,)),
                pl.BlockSpec((w, w), lambda i: (0, 0))],
      out_specs=blk,
      out_shape=jax.ShapeDtypeStruct((m, w), jnp.float32),
  )(sp_pk, g1, rs, b1p, w2blk)


def _layer_out(sp_pk, g2, rs, b2p, n):
  m, w = g2.shape
  h = w // 2
  br = m
  grid = (m // br,)
  scale = 1.0 / n

  def body(sp_ref, g_ref, rs_ref, b_ref, out_ref):
    i = pl.program_id(0)
    a = jnp.maximum(rs_ref[...] * (sp_ref[0] + sp_ref[1] + g_ref[...])
                    + b_ref[...][None, :], 0.0)
    part = jnp.sum(a, axis=0, keepdims=True) * scale
    part = part[:, :h] + part[:, h:]

    @pl.when(i == 0)
    def _():
      out_ref[...] = part

    @pl.when(i > 0)
    def _():
      out_ref[...] += part

  blk = pl.BlockSpec((br, w), lambda i: (i, 0))
  return pl.pallas_call(
      body,
      grid=grid,
      in_specs=[pl.BlockSpec((_NC, br, w), lambda i: (0, i, 0)),
                blk, blk, pl.BlockSpec((w,), lambda i: (0,))],
      out_specs=pl.BlockSpec((1, h), lambda i: (0, 0)),
      out_shape=jax.ShapeDtypeStruct((1, h), jnp.float32),
  )(sp_pk, g2, rs, b2p)


def kernel(x, edge_index, W1, b1, W2, b2):
  n, _ = x.shape
  h = W1.shape[1]
  e = edge_index.shape[1]
  ei_b = lax.optimization_barrier(edge_index)
  src2d = ei_b[0].astype(jnp.int32).reshape(e // _K, _K)
  dst2d = ei_b[1].astype(jnp.int32).reshape(e // _K, _K)
  npad = -(-n // (16 * _NS)) * (16 * _NS)
  zerosh = jnp.zeros((npad, h), jnp.float32)
  dst16 = edge_index[1].astype(jnp.int32).reshape(e // 16, 16)
  df = x.shape[1]
  w1blk = jnp.zeros((2 * df, 2 * h), jnp.float32)
  w1blk = w1blk.at[:df, :h].set(W1).at[df:, h:].set(W1)
  w2blk = jnp.zeros((2 * h, 2 * h), jnp.float32)
  w2blk = w2blk.at[:h, :h].set(W2).at[h:, h:].set(W2)
  b1p = jnp.concatenate([b1, b1])
  b2p = jnp.concatenate([b2, b2])
  xp = x.reshape(n // 2, 2 * df)

  rs_dense = _deg_rs(dst16, npad, h)
  rsp = rs_dense.reshape(npad // 2, 2 * h)
  g1 = _layer_in(xp, rsp, w1blk)
  sp1 = _seg_sum(src2d, dst2d, g1.reshape(n, h), zerosh)
  sp1_pk = sp1.reshape(_NC, npad * h // 128, 128)
  g2 = _layer_mid(sp1_pk, g1, rsp, b1p, w2blk)
  sp2 = _seg_sum(src2d, dst2d, g2.reshape(n, h), zerosh)
  sp2_pk = sp2.reshape(_NC, npad * h // 128, 128)
  out = _layer_out(sp2_pk, g2, rsp, b2p, n)
  return out.reshape(h)


# speedup vs baseline: 1.1109x; 1.1109x over previous
import functools

import jax
import jax.numpy as jnp
from jax import lax
from jax.experimental import pallas as pl
from jax.experimental.pallas import tpu as pltpu
from jax.experimental.pallas import tpu_sc as plsc

_NC = 2
_NS = 16
_K = 125


def _deg_rs(dst16, npad, h):
  erows = dst16.shape[0]
  rpt = npad // _NS
  ept = erows // _NS
  mesh = plsc.VectorSubcoreMesh(core_axis_name="c", subcore_axis_name="s")

  @functools.partial(
      pl.kernel,
      out_type=jax.ShapeDtypeStruct((npad, h), jnp.float32),
      mesh=mesh,
      compiler_params=pltpu.CompilerParams(use_tc_tiling_on_sc=False,
                                           needs_layout_passes=False),
      scratch_types=[
          pltpu.VMEM_SHARED((_NS, npad), jnp.float32),
          pltpu.VMEM((npad,), jnp.float32),
          pltpu.VMEM((ept, 16), jnp.int32),
          pltpu.VMEM((_NS, rpt), jnp.float32),
          pltpu.VMEM((rpt, h), jnp.float32),
          pltpu.SemaphoreType.DMA,
      ],
  )
  def kern(dst_hbm, out_hbm, shared, hist, dstv, sumv, rsb, sem):
    c = lax.axis_index("c")
    s = lax.axis_index("s")
    r0 = s * rpt
    cp = pltpu.async_copy(dst_hbm.at[pl.ds(s * ept, ept)], dstv, sem)

    def zfn(i, carry):
      hist[pl.ds(16 * i, 16)] = jnp.zeros((16,), jnp.float32)
      return carry

    lax.fori_loop(0, npad // 16, zfn, 0)
    cp.wait()

    ones = jnp.ones((16,), jnp.float32)

    def afn(r, carry):
      plsc.addupdate_scatter(hist, [dstv[r]], ones)
      return carry

    lax.fori_loop(0, ept, afn, 0)
    pltpu.sync_copy(hist, shared.at[s])
    plsc.subcore_barrier()

    for t in range(_NS):
      pltpu.async_copy(shared.at[t, pl.ds(r0, rpt)], sumv.at[t], sem)
    for t in range(_NS):
      pltpu.make_async_copy(shared.at[t, pl.ds(r0, rpt)], sumv.at[t],
                            sem).wait()

    def chunk_fn(q, carry):
      col = 16 * q
      tot = sumv[0, pl.ds(col, 16)]
      for t in range(1, _NS):
        tot = tot + sumv[t, pl.ds(col, 16)]
      deg = tot + 1.0
      i = plsc.bitcast(deg, jnp.int32)
      i = jnp.int32(0x5F3759DF) - lax.shift_right_logical(i, 1)
      y = plsc.bitcast(i, jnp.float32)
      y = y * (1.5 - 0.5 * deg * y * y)
      y = y * (1.5 - 0.5 * deg * y * y)
      y = y * (1.5 - 0.5 * deg * y * y)
      for l in range(16):
        val = jnp.full((16,), y[l])
        for q2 in range(h // 16):
          rsb[col + l, pl.ds(16 * q2, 16)] = val
      return carry

    lax.fori_loop(0, rpt // 16, chunk_fn, 0)

    @pl.when(c == 0)
    def _():
      pltpu.sync_copy(rsb, out_hbm.at[pl.ds(r0, rpt)])

  return kern(dst16)


def _seg_sum(src2d, dst2d, g, zeros):
  ec, k = src2d.shape
  ng, d = g.shape
  n = zeros.shape[0]
  rpt = n // _NS
  cpt = ec // (_NC * _NS)
  mesh = plsc.VectorSubcoreMesh(core_axis_name="c", subcore_axis_name="s")

  @functools.partial(
      pl.kernel,
      out_type=jax.ShapeDtypeStruct((_NC, n, d), jnp.float32),
      mesh=mesh,
      compiler_params=pltpu.CompilerParams(use_tc_tiling_on_sc=False),
      scratch_types=[
          pltpu.VMEM_SHARED((n, d), jnp.float32),
          pltpu.VMEM((cpt, k), jnp.int32),
          pltpu.VMEM((cpt, k), jnp.int32),
          [pltpu.VMEM((k, d), jnp.float32)] * 4,
          [pltpu.SemaphoreType.DMA] * 4,
          [pltpu.SemaphoreType.DMA] * 4,
          pltpu.SemaphoreType.DMA,
          pltpu.SemaphoreType.DMA,
      ],
  )
  def kern(src_hbm, dst_hbm, g_hbm, z_hbm, out_hbm, acc, srcv, dstv,
           bufs, gsems, ssems, zsem, isem):
    c = lax.axis_index("c")
    s = lax.axis_index("s")
    r0 = s * rpt
    cz = pltpu.async_copy(z_hbm.at[pl.ds(r0, rpt)], acc.at[pl.ds(r0, rpt)],
                          zsem)
    row0 = c * (ec // _NC) + s * cpt
    cs = pltpu.async_copy(src_hbm.at[pl.ds(row0, cpt)], srcv, isem)
    cd = pltpu.async_copy(dst_hbm.at[pl.ds(row0, cpt)], dstv, isem)
    cs.wait()
    cd.wait()

    def gather(j, b):
      pltpu.async_copy(g_hbm.at[srcv.at[j]], bufs[b], gsems[b])

    def wait_gather(j, b):
      pltpu.make_async_copy(g_hbm.at[srcv.at[j]], bufs[b], gsems[b]).wait()

    def scatter(j, b):
      pltpu.async_copy(bufs[b], acc.at[dstv.at[j]], ssems[b], add=True)

    def wait_scatter(b):
      pltpu.make_async_copy(bufs[b], acc.at[dstv.at[0]], ssems[b]).wait()

    for b in range(4):
      gather(b, b)
    cz.wait()
    plsc.subcore_barrier()

    def body(i, carry):
      for b in range(4):
        j = 4 * i + b
        wait_gather(j, b)
        scatter(j, b)
        bp = (b + 3) % 4
        jprev = j - 1

        @pl.when(jnp.logical_and(jprev >= 0, jprev + 4 < cpt))
        def _():
          wait_scatter(bp)
          gather(jprev + 4, bp)

      return carry

    lax.fori_loop(0, cpt // 4, body, 0)
    for b in range(4):
      wait_scatter(b)
    plsc.subcore_barrier()
    pltpu.sync_copy(acc.at[pl.ds(r0, rpt)], out_hbm.at[c, pl.ds(r0, rpt)])

  return kern(src2d, dst2d, g, zeros)


def _layer_in(xp, rsp, w1blk):
  m, df2 = xp.shape
  w = w1blk.shape[1]
  br = m
  grid = (m // br,)

  def body(x_ref, rs_ref, w_ref, g_ref):
    hh = jnp.dot(x_ref[...], w_ref[...], preferred_element_type=jnp.float32)
    g_ref[...] = hh * rs_ref[...]

  blk = pl.BlockSpec((br, w), lambda i: (i, 0))
  return pl.pallas_call(
      body,
      grid=grid,
      in_specs=[
          pl.BlockSpec((br, df2), lambda i: (i, 0)),
          blk,
          pl.BlockSpec((df2, w), lambda i: (0, 0)),
      ],
      out_specs=blk,
      out_shape=jax.ShapeDtypeStruct((m, w), jnp.float32),
  )(xp, rsp, w1blk)


def _layer_mid(sp_pk, g1, rs, b1p, w2blk):
  m, w = g1.shape
  br = m
  grid = (m // br,)

  def body(sp_ref, g_ref, rs_ref, b_ref, w_ref, out_ref):
    rsv = rs_ref[...]
    a = jnp.maximum(rsv * (sp_ref[0] + sp_ref[1] + g_ref[...])
                    + b_ref[...][None, :], 0.0)
    out_ref[...] = jnp.dot(a, w_ref[...],
                           preferred_element_type=jnp.float32) * rsv

  blk = pl.BlockSpec((br, w), lambda i: (i, 0))
  return pl.pallas_call(
      body,
      grid=grid,
      in_specs=[pl.BlockSpec((_NC, br, w), lambda i: (0, i, 0)),
                blk, blk,
                pl.BlockSpec((w,), lambda i: (0,)),
                pl.BlockSpec((w, w), lambda i: (0, 0))],
      out_specs=blk,
      out_shape=jax.ShapeDtypeStruct((m, w), jnp.float32),
  )(sp_pk, g1, rs, b1p, w2blk)


def _layer_out(sp_pk, g2, rs, b2p, n):
  m, w = g2.shape
  h = w // 2
  br = m
  grid = (m // br,)
  scale = 1.0 / n

  def body(sp_ref, g_ref, rs_ref, b_ref, out_ref):
    i = pl.program_id(0)
    a = jnp.maximum(rs_ref[...] * (sp_ref[0] + sp_ref[1] + g_ref[...])
                    + b_ref[...][None, :], 0.0)
    part = jnp.sum(a, axis=0, keepdims=True) * scale
    part = part[:, :h] + part[:, h:]

    @pl.when(i == 0)
    def _():
      out_ref[...] = part

    @pl.when(i > 0)
    def _():
      out_ref[...] += part

  blk = pl.BlockSpec((br, w), lambda i: (i, 0))
  return pl.pallas_call(
      body,
      grid=grid,
      in_specs=[pl.BlockSpec((_NC, br, w), lambda i: (0, i, 0)),
                blk, blk, pl.BlockSpec((w,), lambda i: (0,))],
      out_specs=pl.BlockSpec((1, h), lambda i: (0, 0)),
      out_shape=jax.ShapeDtypeStruct((1, h), jnp.float32),
  )(sp_pk, g2, rs, b2p)


def kernel(x, edge_index, W1, b1, W2, b2):
  n, _ = x.shape
  h = W1.shape[1]
  e = edge_index.shape[1]
  src2d = edge_index[0].astype(jnp.int32).reshape(e // _K, _K)
  dst2d = edge_index[1].astype(jnp.int32).reshape(e // _K, _K)
  npad = -(-n // (16 * _NS)) * (16 * _NS)
  zerosh = jnp.zeros((npad, h), jnp.float32)
  dst16 = edge_index[1].astype(jnp.int32).reshape(e // 16, 16)
  df = x.shape[1]
  w1blk = jnp.zeros((2 * df, 2 * h), jnp.float32)
  w1blk = w1blk.at[:df, :h].set(W1).at[df:, h:].set(W1)
  w2blk = jnp.zeros((2 * h, 2 * h), jnp.float32)
  w2blk = w2blk.at[:h, :h].set(W2).at[h:, h:].set(W2)
  b1p = jnp.concatenate([b1, b1])
  b2p = jnp.concatenate([b2, b2])
  xp = x.reshape(n // 2, 2 * df)

  rs_dense = _deg_rs(dst16, npad, h)
  rsp = rs_dense.reshape(npad // 2, 2 * h)
  g1 = _layer_in(xp, rsp, w1blk)
  sp1 = _seg_sum(src2d, dst2d, g1.reshape(n, h), zerosh)
  sp1_pk = sp1.reshape(_NC, npad * h // 128, 128)
  g2 = _layer_mid(sp1_pk, g1, rsp, b1p, w2blk)
  sp2 = _seg_sum(src2d, dst2d, g2.reshape(n, h), zerosh)
  sp2_pk = sp2.reshape(_NC, npad * h // 128, 128)
  out = _layer_out(sp2_pk, g2, rsp, b2p, n)
  return out.reshape(h)


# speedup vs baseline: 1.1227x; 1.0107x over previous
import functools

import jax
import jax.numpy as jnp
from jax import lax
from jax.experimental import pallas as pl
from jax.experimental.pallas import tpu as pltpu
from jax.experimental.pallas import tpu_sc as plsc

_NC = 2
_NS = 16
_K = 125


def _deg_rs(dst16, npad, h):
  erows = dst16.shape[0]
  rpt = npad // _NS
  ept = erows // _NS
  mesh = plsc.VectorSubcoreMesh(core_axis_name="c", subcore_axis_name="s")

  @functools.partial(
      pl.kernel,
      out_type=jax.ShapeDtypeStruct((npad, h), jnp.float32),
      mesh=mesh,
      compiler_params=pltpu.CompilerParams(use_tc_tiling_on_sc=False,
                                           needs_layout_passes=False),
      scratch_types=[
          pltpu.VMEM_SHARED((_NS, npad), jnp.float32),
          pltpu.VMEM((npad,), jnp.float32),
          pltpu.VMEM((ept, 16), jnp.int32),
          pltpu.VMEM((_NS, rpt), jnp.float32),
          pltpu.VMEM((rpt, h), jnp.float32),
          pltpu.SemaphoreType.DMA,
      ],
  )
  def kern(dst_hbm, out_hbm, shared, hist, dstv, sumv, rsb, sem):
    c = lax.axis_index("c")
    s = lax.axis_index("s")
    r0 = s * rpt
    cp = pltpu.async_copy(dst_hbm.at[pl.ds(s * ept, ept)], dstv, sem)

    def zfn(i, carry):
      hist[pl.ds(16 * i, 16)] = jnp.zeros((16,), jnp.float32)
      return carry

    lax.fori_loop(0, npad // 16, zfn, 0)
    cp.wait()

    ones = jnp.ones((16,), jnp.float32)

    def afn(r, carry):
      plsc.addupdate_scatter(hist, [dstv[r]], ones)
      return carry

    lax.fori_loop(0, ept, afn, 0)
    pltpu.sync_copy(hist, shared.at[s])
    plsc.subcore_barrier()

    for t in range(_NS):
      pltpu.async_copy(shared.at[t, pl.ds(r0, rpt)], sumv.at[t], sem)
    for t in range(_NS):
      pltpu.make_async_copy(shared.at[t, pl.ds(r0, rpt)], sumv.at[t],
                            sem).wait()

    def chunk_fn(q, carry):
      col = 16 * q
      tot = sumv[0, pl.ds(col, 16)]
      for t in range(1, _NS):
        tot = tot + sumv[t, pl.ds(col, 16)]
      deg = tot + 1.0
      i = plsc.bitcast(deg, jnp.int32)
      i = jnp.int32(0x5F3759DF) - lax.shift_right_logical(i, 1)
      y = plsc.bitcast(i, jnp.float32)
      y = y * (1.5 - 0.5 * deg * y * y)
      y = y * (1.5 - 0.5 * deg * y * y)
      y = y * (1.5 - 0.5 * deg * y * y)
      for l in range(16):
        val = jnp.full((16,), y[l])
        for q2 in range(h // 16):
          rsb[col + l, pl.ds(16 * q2, 16)] = val
      return carry

    lax.fori_loop(0, rpt // 16, chunk_fn, 0)

    half = rpt // 2

    @pl.when(c == 0)
    def _():
      pltpu.sync_copy(rsb.at[pl.ds(0, half)], out_hbm.at[pl.ds(r0, half)])

    @pl.when(c == 1)
    def _():
      pltpu.sync_copy(rsb.at[pl.ds(half, half)],
                      out_hbm.at[pl.ds(r0 + half, half)])

  return kern(dst16)


def _seg_sum(src2d, dst2d, g, zeros):
  ec, k = src2d.shape
  ng, d = g.shape
  n = zeros.shape[0]
  rpt = n // _NS
  cpt = ec // (_NC * _NS)
  mesh = plsc.VectorSubcoreMesh(core_axis_name="c", subcore_axis_name="s")

  @functools.partial(
      pl.kernel,
      out_type=jax.ShapeDtypeStruct((_NC, n, d), jnp.float32),
      mesh=mesh,
      compiler_params=pltpu.CompilerParams(use_tc_tiling_on_sc=False),
      scratch_types=[
          pltpu.VMEM_SHARED((n, d), jnp.float32),
          pltpu.VMEM((cpt, k), jnp.int32),
          pltpu.VMEM((cpt, k), jnp.int32),
          [pltpu.VMEM((k, d), jnp.float32)] * 4,
          [pltpu.SemaphoreType.DMA] * 4,
          [pltpu.SemaphoreType.DMA] * 4,
          pltpu.SemaphoreType.DMA,
          pltpu.SemaphoreType.DMA,
      ],
  )
  def kern(src_hbm, dst_hbm, g_hbm, z_hbm, out_hbm, acc, srcv, dstv,
           bufs, gsems, ssems, zsem, isem):
    c = lax.axis_index("c")
    s = lax.axis_index("s")
    r0 = s * rpt
    cz = pltpu.async_copy(z_hbm.at[pl.ds(r0, rpt)], acc.at[pl.ds(r0, rpt)],
                          zsem)
    row0 = c * (ec // _NC) + s * cpt
    cs = pltpu.async_copy(src_hbm.at[pl.ds(row0, cpt)], srcv, isem)
    cd = pltpu.async_copy(dst_hbm.at[pl.ds(row0, cpt)], dstv, isem)
    cs.wait()
    cd.wait()

    def gather(j, b):
      pltpu.async_copy(g_hbm.at[srcv.at[j]], bufs[b], gsems[b])

    def wait_gather(j, b):
      pltpu.make_async_copy(g_hbm.at[srcv.at[j]], bufs[b], gsems[b]).wait()

    def scatter(j, b):
      pltpu.async_copy(bufs[b], acc.at[dstv.at[j]], ssems[b], add=True)

    def wait_scatter(b):
      pltpu.make_async_copy(bufs[b], acc.at[dstv.at[0]], ssems[b]).wait()

    for b in range(4):
      gather(b, b)
    cz.wait()
    plsc.subcore_barrier()

    def body(i, carry):
      for b in range(4):
        j = 4 * i + b
        wait_gather(j, b)
        scatter(j, b)
        bp = (b + 3) % 4
        jprev = j - 1

        @pl.when(jnp.logical_and(jprev >= 0, jprev + 4 < cpt))
        def _():
          wait_scatter(bp)
          gather(jprev + 4, bp)

      return carry

    lax.fori_loop(0, cpt // 4, body, 0)
    for b in range(4):
      wait_scatter(b)
    plsc.subcore_barrier()
    pltpu.sync_copy(acc.at[pl.ds(r0, rpt)], out_hbm.at[c, pl.ds(r0, rpt)])

  return kern(src2d, dst2d, g, zeros)


def _layer_in(xp, rsp, w1blk):
  m, df2 = xp.shape
  w = w1blk.shape[1]
  br = 2504
  grid = (2,)

  def body(x_ref, rs_ref, w_ref, g_ref):
    hh = jnp.dot(x_ref[...], w_ref[...], preferred_element_type=jnp.float32)
    g_ref[...] = hh * rs_ref[...]

  blk = pl.BlockSpec((br, w), lambda i: (i, 0))
  return pl.pallas_call(
      body,
      grid=grid,
      in_specs=[
          pl.BlockSpec((br, df2), lambda i: (i, 0)),
          blk,
          pl.BlockSpec((df2, w), lambda i: (0, 0)),
      ],
      out_specs=blk,
      out_shape=jax.ShapeDtypeStruct((m, w), jnp.float32),
  )(xp, rsp, w1blk)


def _layer_mid(sp_pk, g1, rs, b1p, w2blk):
  m, w = g1.shape
  br = 2504
  grid = (2,)

  def body(sp_ref, g_ref, rs_ref, b_ref, w_ref, out_ref):
    rsv = rs_ref[...]
    a = jnp.maximum(rsv * (sp_ref[0] + sp_ref[1] + g_ref[...])
                    + b_ref[...][None, :], 0.0)
    out_ref[...] = jnp.dot(a, w_ref[...],
                           preferred_element_type=jnp.float32) * rsv

  blk = pl.BlockSpec((br, w), lambda i: (i, 0))
  return pl.pallas_call(
      body,
      grid=grid,
      in_specs=[pl.BlockSpec((_NC, br, w), lambda i: (0, i, 0)),
                blk, blk,
                pl.BlockSpec((w,), lambda i: (0,)),
                pl.BlockSpec((w, w), lambda i: (0, 0))],
      out_specs=blk,
      out_shape=jax.ShapeDtypeStruct((m, w), jnp.float32),
  )(sp_pk, g1, rs, b1p, w2blk)


def _layer_out(sp_pk, g2, rs, b2p, n):
  m, w = g2.shape
  h = w // 2
  br = m
  grid = (1,)
  scale = 1.0 / n

  def body(sp_ref, g_ref, rs_ref, b_ref, out_ref):
    i = pl.program_id(0)
    a = jnp.maximum(rs_ref[...] * (sp_ref[0] + sp_ref[1] + g_ref[...])
                    + b_ref[...][None, :], 0.0)
    part = jnp.sum(a, axis=0, keepdims=True) * scale
    part = part[:, :h] + part[:, h:]

    @pl.when(i == 0)
    def _():
      out_ref[...] = part

    @pl.when(i > 0)
    def _():
      out_ref[...] += part

  blk = pl.BlockSpec((br, w), lambda i: (i, 0))
  return pl.pallas_call(
      body,
      grid=grid,
      in_specs=[pl.BlockSpec((_NC, br, w), lambda i: (0, i, 0)),
                blk, blk, pl.BlockSpec((w,), lambda i: (0,))],
      out_specs=pl.BlockSpec((1, h), lambda i: (0, 0)),
      out_shape=jax.ShapeDtypeStruct((1, h), jnp.float32),
  )(sp_pk, g2, rs, b2p)


def kernel(x, edge_index, W1, b1, W2, b2):
  n, _ = x.shape
  h = W1.shape[1]
  e = edge_index.shape[1]
  src2d = edge_index[0].astype(jnp.int32).reshape(e // _K, _K)
  dst2d = edge_index[1].astype(jnp.int32).reshape(e // _K, _K)
  npad = -(-n // (16 * _NS)) * (16 * _NS)
  zerosh = jnp.zeros((npad, h), jnp.float32)
  dst16 = edge_index[1].astype(jnp.int32).reshape(e // 16, 16)
  df = x.shape[1]
  w1blk = jnp.zeros((2 * df, 2 * h), jnp.float32)
  w1blk = w1blk.at[:df, :h].set(W1).at[df:, h:].set(W1)
  w2blk = jnp.zeros((2 * h, 2 * h), jnp.float32)
  w2blk = w2blk.at[:h, :h].set(W2).at[h:, h:].set(W2)
  b1p = jnp.concatenate([b1, b1])
  b2p = jnp.concatenate([b2, b2])
  xp = x.reshape(n // 2, 2 * df)

  rs_dense = _deg_rs(dst16, npad, h)
  rsp = rs_dense.reshape(npad // 2, 2 * h)
  g1 = _layer_in(xp, rsp, w1blk)
  sp1 = _seg_sum(src2d, dst2d, g1.reshape(n, h), zerosh)
  sp1_pk = sp1.reshape(_NC, npad * h // 128, 128)
  g2 = _layer_mid(sp1_pk, g1, rsp, b1p, w2blk)
  sp2 = _seg_sum(src2d, dst2d, g2.reshape(n, h), zerosh)
  sp2_pk = sp2.reshape(_NC, npad * h // 128, 128)
  out = _layer_out(sp2_pk, g2, rsp, b2p, n)
  return out.reshape(h)
